# baseline (speedup 1.0000x reference)
import functools

import jax
import jax.numpy as jnp
from jax import lax
from jax.experimental import pallas as pl
from jax.experimental.pallas import tpu as pltpu
from jax.experimental.pallas import tpu_sc as plsc

PAD_ID = 0

_NC = 2
_NS = 16
_NW = _NC * _NS
_G = 128
_NGC = 4
_CH = _G * _NGC


@functools.cache
def _make_sc_gather(n_groups, vocab, d):
  n_rows = n_groups * _G
  assert n_rows % (_NW * _CH) == 0
  g_per_w = n_groups // _NW
  n_chunks = g_per_w // _NGC
  mesh = plsc.VectorSubcoreMesh(core_axis_name="c", subcore_axis_name="s")

  @functools.partial(
      pl.kernel,
      out_type=jax.ShapeDtypeStruct((n_rows, d), jnp.float32),
      mesh=mesh,
      scratch_types=[
          pltpu.VMEM((_NGC, _G), jnp.int32),
          pltpu.VMEM((_CH, d), jnp.float32),
          pltpu.SemaphoreType.DMA,
      ],
      compiler_params=pltpu.CompilerParams(use_tc_tiling_on_sc=False),
  )
  def sc_gather(tok_hbm, table_hbm, out_hbm, idx_v, rows_v, sem):
    wid = lax.axis_index("s") * _NC + lax.axis_index("c")
    g_base = wid * g_per_w

    def chunk_body(c):
      g0 = g_base + c * _NGC
      pltpu.sync_copy(tok_hbm.at[pl.ds(g0, _NGC)], idx_v)
      copies = [
          pltpu.async_copy(
              table_hbm.at[idx_v.at[i]], rows_v.at[pl.ds(i * _G, _G)], sem
          )
          for i in range(_NGC)
      ]
      for cp in copies:
        cp.wait()
      pltpu.sync_copy(rows_v, out_hbm.at[pl.ds(g0 * _G, _CH)])

    pl.loop(0, n_chunks)(chunk_body)

  return sc_gather


def _fixup_body(tok_ref, emb_ref, pos_ref, out_ref):
  t = tok_ref[...]
  e = emb_ref[...]
  seq = e.shape[1]
  p = pos_ref[...][None, :seq, :]
  m = (t != PAD_ID).astype(e.dtype)[..., None]
  out_ref[...] = e * m + p


@functools.cache
def _make_fixup(batch, seq, max_pos, d):
  bb = 64
  assert batch % bb == 0
  return pl.pallas_call(
      _fixup_body,
      grid=(batch // bb,),
      in_specs=[
          pl.BlockSpec((bb, seq), lambda i: (i, 0)),
          pl.BlockSpec((bb, seq, d), lambda i: (i, 0, 0)),
          pl.BlockSpec((max_pos, d), lambda i: (0, 0)),
      ],
      out_specs=pl.BlockSpec((bb, seq, d), lambda i: (i, 0, 0)),
      out_shape=jax.ShapeDtypeStruct((batch, seq, d), jnp.float32),
  )


def kernel(tokens, embed_weight, pos_weight):
  batch, seq = tokens.shape
  vocab, d = embed_weight.shape
  max_pos = pos_weight.shape[0]
  tok32 = tokens.astype(jnp.int32)
  tok_g = tok32.reshape(-1, _G)
  gathered = _make_sc_gather(tok_g.shape[0], vocab, d)(tok_g, embed_weight)
  emb = gathered.reshape(batch, seq, d)
  return _make_fixup(batch, seq, max_pos, d)(tok32, emb, pos_weight)


# speedup vs baseline: 1.0016x; 1.0016x over previous
import functools

import jax
import jax.numpy as jnp
from jax import lax
from jax.experimental import pallas as pl
from jax.experimental.pallas import tpu as pltpu
from jax.experimental.pallas import tpu_sc as plsc

PAD_ID = 0

_NC = 2
_NS = 16
_NW = _NC * _NS
_G = 128
_NGC = 4
_CH = _G * _NGC
_L = 16


@functools.cache
def _make_sc_embed(n_rows, vocab, d, max_pos, seq):
  assert n_rows % (_NW * _CH) == 0 and d % _L == 0
  rows_per_w = n_rows // _NW
  n_chunks = rows_per_w // _CH
  assert rows_per_w % seq == 0
  mesh = plsc.VectorSubcoreMesh(core_axis_name="c", subcore_axis_name="s")

  @functools.partial(
      pl.kernel,
      out_type=jax.ShapeDtypeStruct((n_rows, d), jnp.float32),
      mesh=mesh,
      scratch_types=[
          pltpu.VMEM((_CH,), jnp.int32),
          pltpu.VMEM((_CH,), jnp.int32),
          pltpu.VMEM((_CH, d), jnp.float32),
          pltpu.VMEM((_CH, d), jnp.float32),
          pltpu.VMEM((max_pos, d), jnp.float32),
          pltpu.SemaphoreType.DMA,
          pltpu.SemaphoreType.DMA,
          pltpu.SemaphoreType.DMA,
          pltpu.SemaphoreType.DMA,
      ],
      compiler_params=pltpu.CompilerParams(use_tc_tiling_on_sc=False),
  )
  def sc_embed(tok_hbm, table_hbm, pos_hbm, out_hbm,
               idx0, idx1, rows0, rows1, posbuf, gs0, gs1, ws0, ws1):
    wid = lax.axis_index("s") * _NC + lax.axis_index("c")
    row_base = wid * rows_per_w
    idxb = (idx0, idx1)
    rowsb = (rows0, rows1)
    gsem = (gs0, gs1)
    wsem = (ws0, ws1)

    pltpu.sync_copy(pos_hbm, posbuf)
    _NQ = 4
    _Q = _CH // _NQ

    def fire(c, b):
      r0 = row_base + c * _CH
      pltpu.sync_copy(tok_hbm.at[pl.ds(r0, _CH)], idxb[b])
      for i in range(_NGC):
        pltpu.async_copy(
            table_hbm.at[idxb[b].at[pl.ds(i * _G, _G)]],
            rowsb[b].at[pl.ds(i * _G, _G)],
            gsem[b],
        )

    def drain_gather(b):
      pltpu.make_async_copy(
          table_hbm.at[pl.ds(0, _CH)], rowsb[b], gsem[b]
      ).wait()

    def drain_write(b):
      pltpu.make_async_copy(
          rowsb[b], out_hbm.at[pl.ds(0, _CH)], wsem[b]
      ).wait()

    def fixup_quarter(c, b, q, p0):
      rows = rowsb[b]

      def group_body(g, p):
        tvec = idxb[b][pl.ds(g * _L, _L)]
        mvec = jnp.where(tvec != PAD_ID, 1.0, 0.0).astype(jnp.float32)
        for k in range(_L):
          r = g * _L + k
          m = jnp.full((_L,), mvec[k])
          for j in range(d // _L):
            sl = pl.ds(j * _L, _L)
            rows[r, sl] = rows[r, sl] * m + posbuf[p, sl]
          p = jnp.where(p == seq - 1, 0, p + 1)
        return p

      return lax.fori_loop(q * (_Q // _L), (q + 1) * (_Q // _L),
                           group_body, p0)

    def process(c, b):
      drain_gather(b)
      r0 = row_base + c * _CH
      p = lax.rem(c * _CH, seq)
      for q in range(_NQ):
        p = fixup_quarter(c, b, q, p)
        pltpu.async_copy(
            rowsb[b].at[pl.ds(q * _Q, _Q)],
            out_hbm.at[pl.ds(r0 + q * _Q, _Q)],
            wsem[b],
        )

    fire(0, 0)
    fire(1, 1)
    process(0, 0)

    def steady(cc):
      for b, c in ((1, cc), (0, cc + 1)):
        drain_write(1 - b)
        fire(c + 1, 1 - b)
        process(c, b)

    pl.loop(1, n_chunks - 1, step=2)(steady)
    drain_write(0)
    process(n_chunks - 1, 1)
    drain_write(1)

  return sc_embed


def kernel(tokens, embed_weight, pos_weight):
  batch, seq = tokens.shape
  vocab, d = embed_weight.shape
  max_pos = pos_weight.shape[0]
  tok_flat = tokens.astype(jnp.int32).reshape(-1)
  out = _make_sc_embed(tok_flat.shape[0], vocab, d, max_pos, seq)(
      tok_flat, embed_weight, pos_weight
  )
  return out.reshape(batch, seq, d)
